# baseline (speedup 1.0000x reference)
import functools

import jax
import jax.numpy as jnp
from jax import lax
from jax.experimental import pallas as pl
from jax.experimental.pallas import tpu as pltpu
from jax.experimental.pallas import tpu_sc as plsc

NC = 2
NS = 16
L = 16
CH = 128


@functools.lru_cache(maxsize=None)
def _build(B: int, D: int):
    NW = NC * NS
    D2 = 2 * D
    b_per_w = B // NW
    n_ch = b_per_w // CH
    g_per_ch = CH // L

    mesh = plsc.VectorSubcoreMesh(
        core_axis_name="c", subcore_axis_name="s",
        num_cores=NC, num_subcores=NS)

    @functools.partial(
        pl.kernel,
        out_type=jax.ShapeDtypeStruct((B,), jnp.float32),
        mesh=mesh,
        scratch_types=[
            pltpu.VMEM((b_per_w,), jnp.int32),
            pltpu.VMEM((b_per_w,), jnp.int32),
            pltpu.VMEM((n_ch, CH), jnp.int32),
            pltpu.VMEM((n_ch, CH), jnp.int32),
            pltpu.VMEM((b_per_w,), jnp.int32),
            pltpu.VMEM((b_per_w,), jnp.int32),
            pltpu.VMEM((2, CH, D2), jnp.float32),
            pltpu.VMEM((2, CH, D2), jnp.float32),
            pltpu.VMEM((b_per_w,), jnp.float32),
            pltpu.SemaphoreType.DMA,
            pltpu.SemaphoreType.DMA,
        ],
        compiler_params=pltpu.CompilerParams(needs_layout_passes=False),
    )
    def dis_kernel(src_hbm, dst_hbm, table_hbm, out_hbm,
                   sraw, draw_, sidx, didx, scol, dcol,
                   srows, drows, obuf, sem0, sem1):
        wid = lax.axis_index("s") * NC + lax.axis_index("c")
        base = wid * b_per_w

        pltpu.sync_copy(src_hbm.at[pl.ds(base, b_per_w)], sraw)
        pltpu.sync_copy(dst_hbm.at[pl.ds(base, b_per_w)], draw_)

        for c in range(n_ch):
            for k in range(CH // L):
                sl = pl.ds(c * CH + k * L, L)
                sv = sraw[sl]
                dv = draw_[sl]
                sidx[c, pl.ds(k * L, L)] = sv >> 1
                didx[c, pl.ds(k * L, L)] = dv >> 1
                scol[sl] = (sv & 1) * D
                dcol[sl] = (dv & 1) * D

        sems = (sem0, sem1)

        def fire(c):
            buf = c % 2
            s = sems[buf]
            cp0 = pltpu.async_copy(table_hbm.at[sidx.at[c]], srows.at[buf], s)
            cp1 = pltpu.async_copy(table_hbm.at[didx.at[c]], drows.at[buf], s)
            return (cp0, cp1)

        lane_iota = lax.iota(jnp.int32, L)

        def compute(c):
            buf = c % 2
            sbuf = srows.at[buf]
            dbuf = drows.at[buf]

            def group(g, carry):
                lanes = g * L + lane_iota
                col_s = scol[pl.ds(c * CH + g * L, L)]
                col_d = dcol[pl.ds(c * CH + g * L, L)]
                acc = jnp.zeros((L,), jnp.float32)
                for d in range(D):
                    s = plsc.load_gather(sbuf, [lanes, col_s + d])
                    t = plsc.load_gather(dbuf, [lanes, col_d + d])
                    df = s - t
                    acc = acc + df * df
                x = acc + jnp.float32(1e-12)
                i = plsc.bitcast(x, jnp.int32)
                r = plsc.bitcast(jnp.int32(0x5F3759DF) - (i >> 1),
                                 jnp.float32)
                half_x = jnp.float32(0.5) * x
                for _ in range(3):
                    r = r * (jnp.float32(1.5) - half_x * r * r)
                obuf[pl.ds(c * CH + g * L, L)] = x * r
                return carry

            lax.fori_loop(0, g_per_ch, group, 0)

        inflight = fire(0)
        for c in range(n_ch):
            nxt = fire(c + 1) if c + 1 < n_ch else ()
            for cp in inflight:
                cp.wait()
            compute(c)
            inflight = nxt

        pltpu.sync_copy(obuf, out_hbm.at[pl.ds(base, b_per_w)])

    return dis_kernel


def kernel(input_triplet, table):
    B = input_triplet.shape[0]
    V, D = table.shape
    src = input_triplet[:, 0].astype(jnp.int32)
    dst = input_triplet[:, 1].astype(jnp.int32)
    table2 = table.reshape(V // 2, 2 * D)
    return _build(B, D)(src, dst, table2)
